# baseline (speedup 1.0000x reference)
import jax
import jax.numpy as jnp
from jax import lax
from jax.experimental import pallas as pl
from jax.experimental.pallas import tpu as pltpu
from jax.experimental.pallas import tpu_sc as plsc

_NUM_RANKS = 1024
_MAX_TOK = 77
_D = 768
_CTX = 16
_TPR = 4
_LANES = 16
_NCOL = _D // _LANES

_INFO = plsc.get_sparse_core_info()
_NW = _INFO.num_cores * _INFO.num_subcores
_RPW = _NUM_RANKS // _NW

_TB0 = 16
_NTAIL = _MAX_TOK - _TB0


def _copy_row(dst, dst_row, src, src_row):
    for k in range(_NCOL):
        dst[dst_row, pl.ds(k * _LANES, _LANES)] = (
            src[src_row, pl.ds(k * _LANES, _LANES)])


def _sc_body(ctx_hbm, rank_hbm, sent_hbm, out_hbm, headbuf, t2buf, tailbuf,
             rankbuf, sem):
    wid = lax.axis_index("s") * _INFO.num_cores + lax.axis_index("c")
    base = wid * _RPW

    pltpu.sync_copy(ctx_hbm, tailbuf.at[pl.ds(0, _CTX)])
    for t in range(1, 16):
        _copy_row(headbuf, t, tailbuf, t - 1)
    _copy_row(t2buf, 0, tailbuf, 15)

    def step(i, carry):
        r = base + i
        pltpu.sync_copy(sent_hbm.at[r].at[pl.ds(_TB0, _NTAIL)], tailbuf)
        pltpu.sync_copy(rank_hbm.at[r], rankbuf)
        for j in range(_TPR):
            _copy_row(t2buf, 1 + j, rankbuf, j)
        for j in range(3):
            _copy_row(t2buf, 5 + j, tailbuf, 5 + j)
        pltpu.sync_copy(tailbuf.at[pl.ds(8, _NTAIL - 8)],
                        out_hbm.at[r].at[pl.ds(24, _MAX_TOK - 24)])
        pltpu.sync_copy(t2buf, out_hbm.at[r].at[pl.ds(16, 8)])
        pltpu.sync_copy(sent_hbm.at[r].at[pl.ds(0, 1)],
                        headbuf.at[pl.ds(0, 1)])
        pltpu.sync_copy(headbuf, out_hbm.at[r].at[pl.ds(0, 16)])
        return carry

    lax.fori_loop(0, _RPW, step, 0)


def kernel(context_embeds, rank_embeds, sentence_embeds):
    mesh = plsc.VectorSubcoreMesh(core_axis_name="c", subcore_axis_name="s")
    k = pl.kernel(
        _sc_body,
        out_type=jax.ShapeDtypeStruct((_NUM_RANKS, _MAX_TOK, _D),
                                      jnp.float32),
        mesh=mesh,
        scratch_types=[
            pltpu.VMEM((_CTX, _D), jnp.float32),
            pltpu.VMEM((8, _D), jnp.float32),
            pltpu.VMEM((_NTAIL, _D), jnp.float32),
            pltpu.VMEM((_TPR, _D), jnp.float32),
            pltpu.SemaphoreType.DMA,
        ],
    )
    return k(context_embeds, rank_embeds, sentence_embeds)


# speedup vs baseline: 1.1037x; 1.1037x over previous
import jax
import jax.numpy as jnp
from jax import lax
from jax.experimental import pallas as pl
from jax.experimental.pallas import tpu as pltpu
from jax.experimental.pallas import tpu_sc as plsc

_NUM_RANKS = 1024
_MAX_TOK = 77
_D = 768
_CTX = 16
_TPR = 4
_LANES = 16
_NCOL = _D // _LANES

_INFO = plsc.get_sparse_core_info()
_NW = _INFO.num_cores * _INFO.num_subcores
_RPW = _NUM_RANKS // _NW

_TB0 = 16
_NTAIL = _MAX_TOK - _TB0


def _copy_row(dst, dst_row, src, src_row):
    for k in range(_NCOL):
        dst[dst_row, pl.ds(k * _LANES, _LANES)] = (
            src[src_row, pl.ds(k * _LANES, _LANES)])


def _sc_body(ctx_hbm, rank_hbm, sent_hbm, out_hbm, headbuf, t2buf, tailbuf,
             rankbuf, ti_sem, to_sem, ri_sem, t2_sem, hd_sem, r0_sem):
    wid = lax.axis_index("s") * _INFO.num_cores + lax.axis_index("c")
    base = wid * _RPW

    def tail_in(r, s):
        return pltpu.make_async_copy(
            sent_hbm.at[r].at[pl.ds(_TB0, _NTAIL)], tailbuf.at[s],
            ti_sem.at[s])

    def tail_out(r, s):
        return pltpu.make_async_copy(
            tailbuf.at[s].at[pl.ds(8, _NTAIL - 8)],
            out_hbm.at[r].at[pl.ds(24, _MAX_TOK - 24)], to_sem.at[s])

    def rank_in(r, s):
        return pltpu.make_async_copy(rank_hbm.at[r], rankbuf.at[s],
                                     ri_sem.at[s])

    def t2_out(r):
        return pltpu.make_async_copy(t2buf, out_hbm.at[r].at[pl.ds(16, 8)],
                                     t2_sem)

    def head_out(r):
        return pltpu.make_async_copy(headbuf,
                                     out_hbm.at[r].at[pl.ds(0, 16)], hd_sem)

    def row0_in(r):
        return pltpu.make_async_copy(sent_hbm.at[r].at[pl.ds(0, 1)],
                                     headbuf.at[pl.ds(0, 1)], r0_sem)

    pltpu.sync_copy(ctx_hbm, tailbuf.at[0].at[pl.ds(0, _CTX)])
    for t in range(1, 16):
        _copy_row(headbuf, t, tailbuf.at[0], t - 1)
    _copy_row(t2buf, 0, tailbuf.at[0], 15)

    tail_in(base, 0).start()
    rank_in(base, 0).start()

    def step(i, carry):
        r = base + i
        s = lax.rem(i, 2)

        tail_in(r, s).wait()
        rank_in(r, s).wait()

        @pl.when(i > 0)
        def _free_small():
            t2_out(r - 1).wait()
            head_out(r - 1).wait()

        row0_in(r).start()

        @pl.when(i + 1 < _RPW)
        def _lookahead():
            @pl.when(i > 0)
            def _reclaim():
                tail_out(r - 1, 1 - s).wait()

            tail_in(r + 1, 1 - s).start()
            rank_in(r + 1, 1 - s).start()

        for j in range(_TPR):
            _copy_row(t2buf, 1 + j, rankbuf.at[s], j)
        for j in range(3):
            _copy_row(t2buf, 5 + j, tailbuf.at[s], 5 + j)

        tail_out(r, s).start()
        t2_out(r).start()
        row0_in(r).wait()
        head_out(r).start()
        return carry

    lax.fori_loop(0, _RPW, step, 0)

    last = base + _RPW - 1
    tail_out(last - 1, lax.rem(_RPW - 2, 2)).wait()
    tail_out(last, lax.rem(_RPW - 1, 2)).wait()
    t2_out(last).wait()
    head_out(last).wait()


def kernel(context_embeds, rank_embeds, sentence_embeds):
    mesh = plsc.VectorSubcoreMesh(core_axis_name="c", subcore_axis_name="s")
    k = pl.kernel(
        _sc_body,
        out_type=jax.ShapeDtypeStruct((_NUM_RANKS, _MAX_TOK, _D),
                                      jnp.float32),
        mesh=mesh,
        scratch_types=[
            pltpu.VMEM((_CTX, _D), jnp.float32),
            pltpu.VMEM((8, _D), jnp.float32),
            pltpu.VMEM((2, _NTAIL, _D), jnp.float32),
            pltpu.VMEM((2, _TPR, _D), jnp.float32),
            pltpu.SemaphoreType.DMA((2,)),
            pltpu.SemaphoreType.DMA((2,)),
            pltpu.SemaphoreType.DMA((2,)),
            pltpu.SemaphoreType.DMA,
            pltpu.SemaphoreType.DMA,
            pltpu.SemaphoreType.DMA,
        ],
    )
    return k(context_embeds, rank_embeds, sentence_embeds)


# speedup vs baseline: 3.0744x; 2.7855x over previous
import jax
import jax.numpy as jnp
from jax import lax
from jax.experimental import pallas as pl
from jax.experimental.pallas import tpu as pltpu
from jax.experimental.pallas import tpu_sc as plsc

_NUM_RANKS = 1024
_MAX_TOK = 77
_D = 768
_CTX = 16
_TPR = 4
_LANES = 16
_NCOL = _D // _LANES

_INFO = plsc.get_sparse_core_info()
_NC = _INFO.num_cores
_NS = _INFO.num_subcores
_NW = _NC * _NS
_RPP = _NUM_RANKS // _NW

_T0 = 1 + _CTX + _TPR
_NT = _MAX_TOK - _T0


def _sc_body(ctx_hbm, rank_hbm, sent_hbm, out_hbm, ctxb, sbuf, cbuf,
             si_sem, so_sem, co_sem):
    core = lax.axis_index("c")
    sid = lax.axis_index("s")
    wid = sid * _NC + core
    base = wid * _RPP

    def sin(t, sl):
        return pltpu.make_async_copy(
            sent_hbm.at[t].at[pl.ds(base, _RPP)], sbuf.at[sl],
            si_sem.at[sl])

    def sout(t, sl):
        return pltpu.make_async_copy(
            sbuf.at[sl], out_hbm.at[t].at[pl.ds(base, _RPP)],
            so_sem.at[sl])

    def rin(j, sl):
        return pltpu.make_async_copy(
            rank_hbm.at[j].at[pl.ds(base, _RPP)], sbuf.at[sl],
            si_sem.at[sl])

    def rout(j, sl):
        return pltpu.make_async_copy(
            sbuf.at[sl], out_hbm.at[1 + _CTX + j].at[pl.ds(base, _RPP)],
            so_sem.at[sl])

    def cout(t, sl):
        return pltpu.make_async_copy(
            cbuf.at[sl], out_hbm.at[1 + t].at[pl.ds(base, _RPP)],
            co_sem.at[sl])

    pltpu.sync_copy(ctx_hbm, ctxb)
    sin(_T0, 0).start()

    def step(i, carry):
        t = _T0 + i
        sl = lax.rem(i, 2)

        sin(t, sl).wait()

        @pl.when(i + 1 < _NT)
        def _lookahead():
            @pl.when(i >= 1)
            def _reclaim():
                sout(t - 1, 1 - sl).wait()

            sin(t + 1, 1 - sl).start()

        sout(t, sl).start()

        @pl.when(i < _CTX)
        def _ctx_plane():
            @pl.when(i >= 2)
            def _free():
                cout(i - 2, sl).wait()

            c = cbuf.at[sl]
            for k in range(_NCOL):
                c[0, pl.ds(k * _LANES, _LANES)] = (
                    ctxb[i, pl.ds(k * _LANES, _LANES)])
            for rr in range(1, _RPP):
                for k in range(_NCOL):
                    c[rr, pl.ds(k * _LANES, _LANES)] = (
                        c[0, pl.ds(k * _LANES, _LANES)])
            cout(i, sl).start()

        return carry

    lax.fori_loop(0, _NT, step, 0)

    cout(_CTX - 2, lax.rem(_CTX - 2, 2)).wait()
    cout(_CTX - 1, lax.rem(_CTX - 1, 2)).wait()
    sout(_MAX_TOK - 2, lax.rem(_NT - 2, 2)).wait()
    sout(_MAX_TOK - 1, lax.rem(_NT - 1, 2)).wait()

    def jin(j, sl):
        return sin(0, sl) if j == 0 else rin(j - 1, sl)

    def jout(j, sl):
        return sout(0, sl) if j == 0 else rout(j - 1, sl)

    jin(0, 0).start()
    jin(1, 1).start()
    for j in range(5):
        sl = j % 2
        jin(j, sl).wait()
        jout(j, sl).start()
        if j + 2 < 5:
            jout(j, sl).wait()
            jin(j + 2, sl).start()
    jout(3, 1).wait()
    jout(4, 0).wait()


def kernel(context_embeds, rank_embeds, sentence_embeds):
    sent_t = jnp.transpose(sentence_embeds, (1, 0, 2))
    rank_t = jnp.transpose(rank_embeds, (1, 0, 2))
    mesh = plsc.VectorSubcoreMesh(core_axis_name="c", subcore_axis_name="s")
    k = pl.kernel(
        _sc_body,
        out_type=jax.ShapeDtypeStruct((_MAX_TOK, _NUM_RANKS, _D),
                                      jnp.float32),
        mesh=mesh,
        scratch_types=[
            pltpu.VMEM((_CTX, _D), jnp.float32),
            pltpu.VMEM((2, _RPP, _D), jnp.float32),
            pltpu.VMEM((2, _RPP, _D), jnp.float32),
            pltpu.SemaphoreType.DMA((2,)),
            pltpu.SemaphoreType.DMA((2,)),
            pltpu.SemaphoreType.DMA((2,)),
        ],
    )
    out_t = k(context_embeds, rank_t, sent_t)
    return jnp.transpose(out_t, (1, 0, 2))
